# baseline (speedup 1.0000x reference)
import functools

import jax
import jax.numpy as jnp
from jax import lax
from jax.experimental import pallas as pl
from jax.experimental.pallas import tpu as pltpu
from jax.experimental.pallas import tpu_sc as plsc

_BEAT_LEN = 32
_EMB = 512
_NW = 32
_C = 32
_NBUF = 4
_L = 16


def _sc_body(per_w, n_groups,
             pos_hbm, beat_hbm, bar_hbm, out_hbm,
             pos_v, bidx, btidx, beat_v, rows, sem_g, sem_s):
    wid = lax.axis_index("s") * 2 + lax.axis_index("c")
    base_w = wid * per_w
    n_chunks = n_groups * _NBUF

    pltpu.sync_copy(beat_hbm, beat_v)

    def wait_bar(b):
        pltpu.make_async_copy(bar_hbm.at[bidx[b]], rows[b], sem_g[b]).wait()

    def beat_add(b):
        for q in range(_C // _L):
            btvec = btidx[b][pl.ds(q * _L, _L)]
            tok = lax.iota(jnp.int32, _L) + (q * _L)

            def col_body(c, c2, btvec=btvec, tok=tok):
                colv = jnp.full((_L,), 0, jnp.int32) + c
                vals = plsc.load_gather(beat_v, [btvec, colv])
                plsc.addupdate_scatter(rows[b], [tok, colv], vals)
                return c2

            lax.fori_loop(0, _EMB, col_body, 0, unroll=16)

    def group_body(g, carry):
        base_g = base_w + g * (_NBUF * _C)
        pltpu.sync_copy(pos_hbm.at[pl.ds(base_g, _NBUF * _C)], pos_v)
        for b in range(_NBUF):
            i = g * _NBUF + b

            def drain_store(b=b, i=i):
                st = base_w + (i - _NBUF) * _C
                pltpu.make_async_copy(
                    rows[b], out_hbm.at[pl.ds(st, _C)], sem_s[b]).wait()

            pl.when(g >= 1)(drain_store)

            for q in range(_C // _L):
                src = pl.ds(b * _C + q * _L, _L)
                dst = pl.ds(q * _L, _L)
                p = pos_v[src]
                bidx[b][dst] = lax.shift_right_logical(p, 5)
                btidx[b][dst] = lax.bitwise_and(p, _BEAT_LEN - 1)
            pltpu.async_copy(bar_hbm.at[bidx[b]], rows[b], sem_g[b])

            b1 = (b - 1) % _NBUF

            def finish_prev(b1=b1, i=i):
                wait_bar(b1)
                beat_add(b1)
                st = base_w + (i - 1) * _C
                pltpu.async_copy(rows[b1], out_hbm.at[pl.ds(st, _C)],
                                 sem_s[b1])

            if b == 0:
                pl.when(g >= 1)(finish_prev)
            else:
                finish_prev()
        return carry

    lax.fori_loop(0, n_groups, group_body, 0)

    bl1 = (n_chunks - 1) % _NBUF
    wait_bar(bl1)
    beat_add(bl1)
    pltpu.sync_copy(rows[bl1],
                    out_hbm.at[pl.ds(base_w + (n_chunks - 1) * _C, _C)])
    for j in (4, 3, 2):
        bd = (n_chunks - j) % _NBUF
        st = base_w + (n_chunks - j) * _C
        pltpu.make_async_copy(
            rows[bd], out_hbm.at[pl.ds(st, _C)], sem_s[bd]).wait()


def kernel(pos, beat_W, bar_W):
    b, s = pos.shape
    n = b * s
    per_w = n // _NW
    n_groups = per_w // (_C * _NBUF)
    assert per_w * _NW == n and n_groups * _C * _NBUF == per_w

    pos_flat = pos.reshape(n)
    beat_w0 = beat_W.at[0].set(0.0)
    bar_w0 = bar_W.at[0].set(0.0)

    mesh = plsc.VectorSubcoreMesh(core_axis_name="c", subcore_axis_name="s")

    def body(pos_hbm, beat_hbm, bar_hbm, out_hbm, pos_v, beat_v, *bufs):
        bidx = bufs[0:_NBUF]
        btidx = bufs[_NBUF:2 * _NBUF]
        rows = bufs[2 * _NBUF:3 * _NBUF]
        sem_g = bufs[3 * _NBUF:4 * _NBUF]
        sem_s = bufs[4 * _NBUF:5 * _NBUF]
        _sc_body(per_w, n_groups, pos_hbm, beat_hbm, bar_hbm, out_hbm,
                 pos_v, bidx, btidx, beat_v, rows, sem_g, sem_s)

    run = pl.kernel(
        body,
        out_type=jax.ShapeDtypeStruct((n, _EMB), jnp.float32),
        mesh=mesh,
        compiler_params=pltpu.CompilerParams(
            use_tc_tiling_on_sc=False, needs_layout_passes=False),
        scratch_types=(
            [pltpu.VMEM((_NBUF * _C,), jnp.int32),
             pltpu.VMEM((_BEAT_LEN, _EMB), jnp.float32)]
            + [pltpu.VMEM((_C,), jnp.int32) for _ in range(2 * _NBUF)]
            + [pltpu.VMEM((_C, _EMB), jnp.float32) for _ in range(_NBUF)]
            + [pltpu.SemaphoreType.DMA for _ in range(2 * _NBUF)]
        ),
    )
    out = run(pos_flat, beat_w0, bar_w0)
    return out.reshape(b, s, _EMB)


# speedup vs baseline: 6.2232x; 6.2232x over previous
import functools

import jax
import jax.numpy as jnp
from jax import lax
from jax.experimental import pallas as pl
from jax.experimental.pallas import tpu as pltpu
from jax.experimental.pallas import tpu_sc as plsc

_BEAT_LEN = 32
_EMB = 512
_NW = 32
_C = 32
_NBUF = 4
_L = 16


def _sc_body(per_w, n_groups,
             pos_hbm, beat_hbm, bar_hbm, out_hbm,
             pos_v, bidx, btidx, beat_v, rows, sem_g, sem_s):
    wid = lax.axis_index("s") * 2 + lax.axis_index("c")
    base_w = wid * per_w
    n_chunks = n_groups * _NBUF

    @pl.when(lax.axis_index("s") == 0)
    def _stage_beat():
        pltpu.sync_copy(beat_hbm, beat_v)

    plsc.subcore_barrier()

    def wait_bar(b):
        pltpu.make_async_copy(bar_hbm.at[bidx[b]], rows[b], sem_g[b]).wait()

    def beat_add(b):
        pltpu.async_copy(beat_v.at[btidx[b]], rows[b], sem_g[b], add=True)
        pltpu.make_async_copy(beat_v.at[btidx[b]], rows[b], sem_g[b]).wait()

    def group_body(g, carry):
        base_g = base_w + g * (_NBUF * _C)
        pltpu.sync_copy(pos_hbm.at[pl.ds(base_g, _NBUF * _C)], pos_v)
        for b in range(_NBUF):
            i = g * _NBUF + b

            def drain_store(b=b, i=i):
                st = base_w + (i - _NBUF) * _C
                pltpu.make_async_copy(
                    rows[b], out_hbm.at[pl.ds(st, _C)], sem_s[b]).wait()

            pl.when(g >= 1)(drain_store)

            for q in range(_C // _L):
                src = pl.ds(b * _C + q * _L, _L)
                dst = pl.ds(q * _L, _L)
                p = pos_v[src]
                bidx[b][dst] = lax.shift_right_logical(p, 5)
                btidx[b][dst] = lax.bitwise_and(p, _BEAT_LEN - 1)
            pltpu.async_copy(bar_hbm.at[bidx[b]], rows[b], sem_g[b])

            b1 = (b - 1) % _NBUF

            def finish_prev(b1=b1, i=i):
                wait_bar(b1)
                beat_add(b1)
                st = base_w + (i - 1) * _C
                pltpu.async_copy(rows[b1], out_hbm.at[pl.ds(st, _C)],
                                 sem_s[b1])

            if b == 0:
                pl.when(g >= 1)(finish_prev)
            else:
                finish_prev()
        return carry

    lax.fori_loop(0, n_groups, group_body, 0)

    bl1 = (n_chunks - 1) % _NBUF
    wait_bar(bl1)
    beat_add(bl1)
    pltpu.sync_copy(rows[bl1],
                    out_hbm.at[pl.ds(base_w + (n_chunks - 1) * _C, _C)])
    for j in (4, 3, 2):
        bd = (n_chunks - j) % _NBUF
        st = base_w + (n_chunks - j) * _C
        pltpu.make_async_copy(
            rows[bd], out_hbm.at[pl.ds(st, _C)], sem_s[bd]).wait()


def kernel(pos, beat_W, bar_W):
    b, s = pos.shape
    n = b * s
    per_w = n // _NW
    n_groups = per_w // (_C * _NBUF)
    assert per_w * _NW == n and n_groups * _C * _NBUF == per_w

    pos_flat = pos.reshape(n)
    beat_w0 = beat_W.at[0].set(0.0)
    bar_w0 = bar_W.at[0].set(0.0)

    mesh = plsc.VectorSubcoreMesh(core_axis_name="c", subcore_axis_name="s")

    def body(pos_hbm, beat_hbm, bar_hbm, out_hbm, pos_v, beat_v, *bufs):
        bidx = bufs[0:_NBUF]
        btidx = bufs[_NBUF:2 * _NBUF]
        rows = bufs[2 * _NBUF:3 * _NBUF]
        sem_g = bufs[3 * _NBUF:4 * _NBUF]
        sem_s = bufs[4 * _NBUF:5 * _NBUF]
        _sc_body(per_w, n_groups, pos_hbm, beat_hbm, bar_hbm, out_hbm,
                 pos_v, bidx, btidx, beat_v, rows, sem_g, sem_s)

    run = pl.kernel(
        body,
        out_type=jax.ShapeDtypeStruct((n, _EMB), jnp.float32),
        mesh=mesh,
        compiler_params=pltpu.CompilerParams(
            use_tc_tiling_on_sc=False, needs_layout_passes=False),
        scratch_types=(
            [pltpu.VMEM((_NBUF * _C,), jnp.int32),
             pltpu.VMEM_SHARED((_BEAT_LEN, _EMB), jnp.float32)]
            + [pltpu.VMEM((_C,), jnp.int32) for _ in range(2 * _NBUF)]
            + [pltpu.VMEM((_C, _EMB), jnp.float32) for _ in range(_NBUF)]
            + [pltpu.SemaphoreType.DMA for _ in range(2 * _NBUF)]
        ),
    )
    out = run(pos_flat, beat_w0, bar_w0)
    return out.reshape(b, s, _EMB)


# speedup vs baseline: 6.3038x; 1.0129x over previous
import functools

import jax
import jax.numpy as jnp
from jax import lax
from jax.experimental import pallas as pl
from jax.experimental.pallas import tpu as pltpu
from jax.experimental.pallas import tpu_sc as plsc

_BEAT_LEN = 32
_EMB = 512
_NW = 32
_C = 32
_NBUF = 4
_L = 16


def _sc_body(per_w, n_groups,
             pos_hbm, beat_hbm, bar_hbm, out_hbm,
             pos_v, bidx, btidx, beat_v, rows, sem_g, sem_s):
    wid = lax.axis_index("s") * 2 + lax.axis_index("c")
    base_w = wid * per_w
    n_chunks = n_groups * _NBUF

    @pl.when(lax.axis_index("s") == 0)
    def _stage_beat():
        pltpu.sync_copy(beat_hbm, beat_v)

    plsc.subcore_barrier()

    def wait_bar(b):
        pltpu.make_async_copy(bar_hbm.at[bidx[b]], rows[b], sem_g[b]).wait()

    def beat_add(b):
        pltpu.async_copy(beat_v.at[btidx[b]], rows[b], sem_g[b], add=True)
        pltpu.make_async_copy(beat_v.at[btidx[b]], rows[b], sem_g[b]).wait()

    pltpu.sync_copy(pos_hbm.at[pl.ds(base_w, per_w)], pos_v)

    def group_body(g, carry):
        for b in range(_NBUF):
            i = g * _NBUF + b

            def drain_store(b=b, i=i):
                st = base_w + (i - _NBUF) * _C
                pltpu.make_async_copy(
                    rows[b], out_hbm.at[pl.ds(st, _C)], sem_s[b]).wait()

            pl.when(g >= 1)(drain_store)

            for q in range(_C // _L):
                src = pl.ds(i * _C + q * _L, _L)
                dst = pl.ds(q * _L, _L)
                p = pos_v[src]
                bidx[b][dst] = lax.shift_right_logical(p, 5)
                btidx[b][dst] = lax.bitwise_and(p, _BEAT_LEN - 1)
            pltpu.async_copy(bar_hbm.at[bidx[b]], rows[b], sem_g[b])

            b1 = (b - 1) % _NBUF

            def finish_prev(b1=b1, i=i):
                wait_bar(b1)
                beat_add(b1)
                st = base_w + (i - 1) * _C
                pltpu.async_copy(rows[b1], out_hbm.at[pl.ds(st, _C)],
                                 sem_s[b1])

            if b == 0:
                pl.when(g >= 1)(finish_prev)
            else:
                finish_prev()
        return carry

    lax.fori_loop(0, n_groups, group_body, 0)

    bl1 = (n_chunks - 1) % _NBUF
    wait_bar(bl1)
    beat_add(bl1)
    pltpu.sync_copy(rows[bl1],
                    out_hbm.at[pl.ds(base_w + (n_chunks - 1) * _C, _C)])
    for j in (4, 3, 2):
        bd = (n_chunks - j) % _NBUF
        st = base_w + (n_chunks - j) * _C
        pltpu.make_async_copy(
            rows[bd], out_hbm.at[pl.ds(st, _C)], sem_s[bd]).wait()


def kernel(pos, beat_W, bar_W):
    b, s = pos.shape
    n = b * s
    per_w = n // _NW
    n_groups = per_w // (_C * _NBUF)
    assert per_w * _NW == n and n_groups * _C * _NBUF == per_w

    pos_flat = pos.reshape(n)
    beat_w0 = beat_W.at[0].set(0.0)
    bar_w0 = bar_W.at[0].set(0.0)

    mesh = plsc.VectorSubcoreMesh(core_axis_name="c", subcore_axis_name="s")

    def body(pos_hbm, beat_hbm, bar_hbm, out_hbm, pos_v, beat_v, *bufs):
        bidx = bufs[0:_NBUF]
        btidx = bufs[_NBUF:2 * _NBUF]
        rows = bufs[2 * _NBUF:3 * _NBUF]
        sem_g = bufs[3 * _NBUF:4 * _NBUF]
        sem_s = bufs[4 * _NBUF:5 * _NBUF]
        _sc_body(per_w, n_groups, pos_hbm, beat_hbm, bar_hbm, out_hbm,
                 pos_v, bidx, btidx, beat_v, rows, sem_g, sem_s)

    run = pl.kernel(
        body,
        out_type=jax.ShapeDtypeStruct((n, _EMB), jnp.float32),
        mesh=mesh,
        compiler_params=pltpu.CompilerParams(
            use_tc_tiling_on_sc=False, needs_layout_passes=False),
        scratch_types=(
            [pltpu.VMEM((per_w,), jnp.int32),
             pltpu.VMEM_SHARED((_BEAT_LEN, _EMB), jnp.float32)]
            + [pltpu.VMEM((_C,), jnp.int32) for _ in range(2 * _NBUF)]
            + [pltpu.VMEM((_C, _EMB), jnp.float32) for _ in range(_NBUF)]
            + [pltpu.SemaphoreType.DMA for _ in range(2 * _NBUF)]
        ),
    )
    out = run(pos_flat, beat_w0, bar_w0)
    return out.reshape(b, s, _EMB)


# speedup vs baseline: 6.3056x; 1.0003x over previous
import functools

import jax
import jax.numpy as jnp
from jax import lax
from jax.experimental import pallas as pl
from jax.experimental.pallas import tpu as pltpu
from jax.experimental.pallas import tpu_sc as plsc

_BEAT_LEN = 32
_EMB = 512
_NW = 32
_C = 80
_NBUF = 2
_L = 16


def _sc_body(per_w, n_groups,
             pos_hbm, beat_hbm, bar_hbm, out_hbm,
             pos_v, bidx, btidx, beat_v, rows, sem_g, sem_s):
    wid = lax.axis_index("s") * 2 + lax.axis_index("c")
    base_w = wid * per_w
    n_chunks = n_groups * _NBUF

    @pl.when(lax.axis_index("s") == 0)
    def _stage_beat():
        pltpu.sync_copy(beat_hbm, beat_v)

    plsc.subcore_barrier()

    def wait_bar(b):
        pltpu.make_async_copy(bar_hbm.at[bidx[b]], rows[b], sem_g[b]).wait()

    def beat_add(b):
        pltpu.async_copy(beat_v.at[btidx[b]], rows[b], sem_g[b], add=True)
        pltpu.make_async_copy(beat_v.at[btidx[b]], rows[b], sem_g[b]).wait()

    pltpu.sync_copy(pos_hbm.at[pl.ds(base_w, per_w)], pos_v)

    def group_body(g, carry):
        for b in range(_NBUF):
            i = g * _NBUF + b

            def drain_store(b=b, i=i):
                st = base_w + (i - _NBUF) * _C
                pltpu.make_async_copy(
                    rows[b], out_hbm.at[pl.ds(st, _C)], sem_s[b]).wait()

            pl.when(g >= 1)(drain_store)

            for q in range(_C // _L):
                src = pl.ds(i * _C + q * _L, _L)
                dst = pl.ds(q * _L, _L)
                p = pos_v[src]
                bidx[b][dst] = lax.shift_right_logical(p, 5)
                btidx[b][dst] = lax.bitwise_and(p, _BEAT_LEN - 1)
            pltpu.async_copy(bar_hbm.at[bidx[b]], rows[b], sem_g[b])

            b1 = (b - 1) % _NBUF

            def finish_prev(b1=b1, i=i):
                wait_bar(b1)
                beat_add(b1)
                st = base_w + (i - 1) * _C
                pltpu.async_copy(rows[b1], out_hbm.at[pl.ds(st, _C)],
                                 sem_s[b1])

            if b == 0:
                pl.when(g >= 1)(finish_prev)
            else:
                finish_prev()
        return carry

    lax.fori_loop(0, n_groups, group_body, 0)

    bl1 = (n_chunks - 1) % _NBUF
    wait_bar(bl1)
    beat_add(bl1)
    pltpu.sync_copy(rows[bl1],
                    out_hbm.at[pl.ds(base_w + (n_chunks - 1) * _C, _C)])
    for j in range(_NBUF, 1, -1):
        bd = (n_chunks - j) % _NBUF
        st = base_w + (n_chunks - j) * _C
        pltpu.make_async_copy(
            rows[bd], out_hbm.at[pl.ds(st, _C)], sem_s[bd]).wait()


def kernel(pos, beat_W, bar_W):
    b, s = pos.shape
    n = b * s
    per_w = n // _NW
    n_groups = per_w // (_C * _NBUF)
    assert per_w * _NW == n and n_groups * _C * _NBUF == per_w

    pos_flat = pos.reshape(n)
    beat_w0 = beat_W.at[0].set(0.0)
    bar_w0 = bar_W.at[0].set(0.0)

    mesh = plsc.VectorSubcoreMesh(core_axis_name="c", subcore_axis_name="s")

    def body(pos_hbm, beat_hbm, bar_hbm, out_hbm, pos_v, beat_v, *bufs):
        bidx = bufs[0:_NBUF]
        btidx = bufs[_NBUF:2 * _NBUF]
        rows = bufs[2 * _NBUF:3 * _NBUF]
        sem_g = bufs[3 * _NBUF:4 * _NBUF]
        sem_s = bufs[4 * _NBUF:5 * _NBUF]
        _sc_body(per_w, n_groups, pos_hbm, beat_hbm, bar_hbm, out_hbm,
                 pos_v, bidx, btidx, beat_v, rows, sem_g, sem_s)

    run = pl.kernel(
        body,
        out_type=jax.ShapeDtypeStruct((n, _EMB), jnp.float32),
        mesh=mesh,
        compiler_params=pltpu.CompilerParams(
            use_tc_tiling_on_sc=False, needs_layout_passes=False),
        scratch_types=(
            [pltpu.VMEM((per_w,), jnp.int32),
             pltpu.VMEM_SHARED((_BEAT_LEN, _EMB), jnp.float32)]
            + [pltpu.VMEM((_C,), jnp.int32) for _ in range(2 * _NBUF)]
            + [pltpu.VMEM((_C, _EMB), jnp.float32) for _ in range(_NBUF)]
            + [pltpu.SemaphoreType.DMA for _ in range(2 * _NBUF)]
        ),
    )
    out = run(pos_flat, beat_w0, bar_w0)
    return out.reshape(b, s, _EMB)
